# Initial kernel scaffold; baseline (speedup 1.0000x reference)
import jax
import jax.numpy as jnp
from jax.experimental import pallas as pl


def kernel(input, indices):
    raise NotImplementedError("write your pallas kernel here")


# speedup vs baseline: 1.8725x; 1.8725x over previous
import functools

import jax
import jax.numpy as jnp
from jax import lax
from jax.experimental import pallas as pl
from jax.experimental.pallas import tpu as pltpu
from jax.experimental.pallas import tpu_sc as plsc

D = 64
CHUNK = 128
GPB = 4
ROWS = CHUNK * GPB


def _build(N, NC, NS):
    NW = NC * NS
    per_w = N // NW
    rows_per_w = per_w // CHUNK
    outer = per_w // ROWS
    assert per_w * NW == N and rows_per_w * CHUNK == per_w
    assert outer * ROWS == per_w and outer % 2 == 0

    mesh = plsc.VectorSubcoreMesh(core_axis_name="c", subcore_axis_name="s")

    @functools.partial(
        pl.kernel,
        out_type=jax.ShapeDtypeStruct((N, D), jnp.float32),
        mesh=mesh,
        compiler_params=pltpu.CompilerParams(use_tc_tiling_on_sc=False),
        scratch_types=[
            pltpu.VMEM((rows_per_w, CHUNK), jnp.int32),
            pltpu.VMEM((ROWS, D), jnp.float32),
            pltpu.VMEM((ROWS, D), jnp.float32),
            pltpu.SemaphoreType.DMA,
            pltpu.SemaphoreType.DMA,
            pltpu.SemaphoreType.DMA,
        ],
    )
    def gather_kernel(table_hbm, idx_hbm, out_hbm, idx_v, rows0, rows1,
                      gsem, osem0, osem1):
        wid = lax.axis_index("s") * NC + lax.axis_index("c")
        idx_row0 = wid * rows_per_w
        out_base = wid * per_w

        pltpu.sync_copy(idx_hbm.at[pl.ds(idx_row0, rows_per_w)], idx_v)

        def fill(rows, s):
            for g in range(GPB):
                j = s * GPB + g
                pltpu.make_async_copy(
                    table_hbm.at[idx_v.at[j]],
                    rows.at[pl.ds(g * CHUNK, CHUNK)],
                    gsem,
                ).start()
            pltpu.make_async_copy(out_hbm.at[pl.ds(0, ROWS)], rows, gsem).wait()

        def flush(rows, osem, s):
            pltpu.make_async_copy(
                rows, out_hbm.at[pl.ds(out_base + s * ROWS, ROWS)], osem
            ).start()

        def wait_flush(rows, osem):
            pltpu.make_async_copy(rows, out_hbm.at[pl.ds(out_base, ROWS)],
                                  osem).wait()

        def body(o, carry):
            for b, (rows, osem) in enumerate(((rows0, osem0), (rows1, osem1))):
                s = 2 * o + b

                @pl.when(s >= 2)
                def _():
                    wait_flush(rows, osem)

                fill(rows, s)
                flush(rows, osem, s)
            return carry

        lax.fori_loop(0, outer // 2, body, 0)
        wait_flush(rows0, osem0)
        wait_flush(rows1, osem1)

    return gather_kernel


def kernel(input, indices):
    B, K = indices.shape
    N = B * K
    info = plsc.get_sparse_core_info()
    NC, NS = info.num_cores, info.num_subcores
    idx2d = indices.reshape(N // CHUNK, CHUNK).astype(jnp.int32)
    out = _build(N, NC, NS)(input, idx2d)
    return out.reshape(B, K, D)


# speedup vs baseline: 1.9925x; 1.0641x over previous
import functools

import jax
import jax.numpy as jnp
from jax import lax
from jax.experimental import pallas as pl
from jax.experimental.pallas import tpu as pltpu
from jax.experimental.pallas import tpu_sc as plsc

D = 64
CHUNK = 256
GPB = 2
ROWS = CHUNK * GPB


def _build(N, NC, NS):
    NW = NC * NS
    per_w = N // NW
    rows_per_w = per_w // CHUNK
    outer = per_w // ROWS
    assert per_w * NW == N and rows_per_w * CHUNK == per_w
    assert outer * ROWS == per_w and outer % 2 == 0

    mesh = plsc.VectorSubcoreMesh(core_axis_name="c", subcore_axis_name="s")

    @functools.partial(
        pl.kernel,
        out_type=jax.ShapeDtypeStruct((N, D), jnp.float32),
        mesh=mesh,
        compiler_params=pltpu.CompilerParams(use_tc_tiling_on_sc=False),
        scratch_types=[
            pltpu.VMEM((rows_per_w, CHUNK), jnp.int32),
            pltpu.VMEM((ROWS, D), jnp.float32),
            pltpu.VMEM((ROWS, D), jnp.float32),
            pltpu.SemaphoreType.DMA,
            pltpu.SemaphoreType.DMA,
            pltpu.SemaphoreType.DMA,
        ],
    )
    def gather_kernel(table_hbm, idx_hbm, out_hbm, idx_v, rows0, rows1,
                      gsem, osem0, osem1):
        wid = lax.axis_index("s") * NC + lax.axis_index("c")
        idx_row0 = wid * rows_per_w
        out_base = wid * per_w

        pltpu.sync_copy(idx_hbm.at[pl.ds(idx_row0, rows_per_w)], idx_v)

        def fill(rows, s):
            for g in range(GPB):
                j = s * GPB + g
                pltpu.make_async_copy(
                    table_hbm.at[idx_v.at[j]],
                    rows.at[pl.ds(g * CHUNK, CHUNK)],
                    gsem,
                ).start()
            pltpu.make_async_copy(out_hbm.at[pl.ds(0, ROWS)], rows, gsem).wait()

        def flush(rows, osem, s):
            pltpu.make_async_copy(
                rows, out_hbm.at[pl.ds(out_base + s * ROWS, ROWS)], osem
            ).start()

        def wait_flush(rows, osem):
            pltpu.make_async_copy(rows, out_hbm.at[pl.ds(out_base, ROWS)],
                                  osem).wait()

        def body(o, carry):
            for b, (rows, osem) in enumerate(((rows0, osem0), (rows1, osem1))):
                s = 2 * o + b

                @pl.when(s >= 2)
                def _():
                    wait_flush(rows, osem)

                fill(rows, s)
                flush(rows, osem, s)
            return carry

        lax.fori_loop(0, outer // 2, body, 0)
        wait_flush(rows0, osem0)
        wait_flush(rows1, osem1)

    return gather_kernel


def kernel(input, indices):
    B, K = indices.shape
    N = B * K
    info = plsc.get_sparse_core_info()
    NC, NS = info.num_cores, info.num_subcores
    idx2d = indices.reshape(N // CHUNK, CHUNK).astype(jnp.int32)
    out = _build(N, NC, NS)(input, idx2d)
    return out.reshape(B, K, D)
